# Initial kernel scaffold; baseline (speedup 1.0000x reference)
import jax
import jax.numpy as jnp
from jax.experimental import pallas as pl


def kernel(x, edge_index, edge_attr, Wq1, bq1, Wk1, bk1, Wv1, bv1, We1, Ws1, bs1, Wq2, bq2, Wk2, bk2, Wv2, bv2, We2, Ws2, bs2):
    raise NotImplementedError("write your pallas kernel here")


# speedup vs baseline: 20.9334x; 20.9334x over previous
import functools

import jax
import jax.numpy as jnp
from jax import lax
from jax.experimental import pallas as pl
from jax.experimental.pallas import tpu as pltpu
from jax.experimental.pallas import tpu_sc as plsc

NN = 50000
EE = 1600000
CH = 128
NCHUNK = EE // CH
NW = 32
ROWS_PER_TILE = 3128
NPAD = 16 * ROWS_PER_TILE
ZR = 184
ZCOPIES = ROWS_PER_TILE // ZR

_mesh = plsc.VectorSubcoreMesh(core_axis_name="c", subcore_axis_name="s")
_params = pltpu.CompilerParams(needs_layout_passes=False,
                               use_tc_tiling_on_sc=False)


def _full(val):
    return jnp.full((16,), val, jnp.int32)


@functools.partial(
    pl.kernel,
    out_type=jax.ShapeDtypeStruct((2, NPAD, 32), jnp.float32),
    mesh=_mesh,
    scratch_types=[
        pltpu.VMEM((CH,), jnp.int32),
        pltpu.VMEM((CH,), jnp.int32),
        pltpu.VMEM((CH, 16), jnp.float32),
        pltpu.VMEM((CH, 16), jnp.float32),
        pltpu.VMEM((CH, 32), jnp.float32),
        pltpu.VMEM((CH, 32), jnp.float32),
        pltpu.VMEM((ZR, 32), jnp.float32),
        pltpu.VMEM_SHARED((NPAD, 32), jnp.float32),
        pltpu.SemaphoreType.DMA,
        pltpu.SemaphoreType.DMA,
    ],
    compiler_params=_params,
)
def _edge_pass1(q_hbm, kv_hbm, src_hbm, dst_hbm, e_hbm, out_hbm,
                src_v, dst_v, e_v, q_v, kv_v, msg_v, zb_v, acc_sh,
                sem1, sem2):
    c = lax.axis_index("c")
    s = lax.axis_index("s")
    wid = s * 2 + c

    z16 = jnp.zeros((16,), jnp.float32)

    def _zero_zb(i, carry):
        zb_v[i, pl.ds(0, 16)] = z16
        zb_v[i, pl.ds(16, 16)] = z16
        return carry

    lax.fori_loop(0, ZR, _zero_zb, 0)

    def _zero_msg(i, carry):
        msg_v[i, pl.ds(0, 16)] = z16
        msg_v[i, pl.ds(16, 16)] = z16
        return carry

    lax.fori_loop(0, CH, _zero_msg, 0)
    for i in range(ZCOPIES):
        pltpu.sync_copy(zb_v, acc_sh.at[pl.ds(s * ROWS_PER_TILE + i * ZR, ZR)])
    plsc.subcore_barrier()

    nch = 390 + jnp.where(wid < NCHUNK - 390 * NW, 1, 0)
    iota = lax.iota(jnp.int32, 16)

    def _chunk(j, carry):
        base = (wid + j * NW) * CH
        pltpu.sync_copy(src_hbm.at[pl.ds(base, CH)], src_v)
        pltpu.sync_copy(dst_hbm.at[pl.ds(base, CH)], dst_v)
        pltpu.sync_copy(e_hbm.at[pl.ds(base, CH)], e_v)
        cp1 = pltpu.async_copy(q_hbm.at[dst_v], q_v, sem1)
        cp2 = pltpu.async_copy(kv_hbm.at[src_v], kv_v, sem2)
        cp1.wait()
        cp2.wait()
        for g in range(CH // 16):
            lanes = iota + (g * 16)
            acc = jnp.zeros((16,), jnp.float32)
            ecols = []
            for f in range(16):
                qf = plsc.load_gather(q_v, [lanes, _full(f)])
                kf = plsc.load_gather(kv_v, [lanes, _full(f)])
                ef = plsc.load_gather(e_v, [lanes, _full(f)])
                ecols.append(ef)
                acc = acc + qf * (kf + ef)
            alpha = jnp.clip(acc * 0.25, -75.0, 75.0)
            w = jnp.exp(alpha)
            for f in range(16):
                vf = plsc.load_gather(kv_v, [lanes, _full(16 + f)])
                plsc.store_scatter(msg_v, [lanes, _full(f)],
                                   w * (vf + ecols[f]))
            plsc.store_scatter(msg_v, [lanes, _full(16)], w)
        pltpu.sync_copy(msg_v, acc_sh.at[dst_v], add=True)
        return carry

    lax.fori_loop(0, nch, _chunk, 0)
    plsc.subcore_barrier()
    r0 = s * ROWS_PER_TILE
    pltpu.sync_copy(acc_sh.at[pl.ds(r0, ROWS_PER_TILE)],
                    out_hbm.at[c, pl.ds(r0, ROWS_PER_TILE)])


@functools.partial(
    pl.kernel,
    out_type=jax.ShapeDtypeStruct((2, NPAD, 16), jnp.float32),
    mesh=_mesh,
    scratch_types=[
        pltpu.VMEM((CH,), jnp.int32),
        pltpu.VMEM((CH,), jnp.int32),
        pltpu.VMEM((CH,), jnp.float32),
        pltpu.VMEM((CH, 16), jnp.float32),
        pltpu.VMEM((CH, 16), jnp.float32),
        pltpu.VMEM((CH, 16), jnp.float32),
        pltpu.VMEM((ZR, 16), jnp.float32),
        pltpu.VMEM_SHARED((NPAD, 16), jnp.float32),
        pltpu.SemaphoreType.DMA,
        pltpu.SemaphoreType.DMA,
    ],
    compiler_params=_params,
)
def _edge_pass2(t2_hbm, src_hbm, dst_hbm, e2_hbm, out_hbm,
                src_v, dst_v, e2_v, ts_v, td_v, msg_v, zb_v, acc_sh,
                sem1, sem2):
    c = lax.axis_index("c")
    s = lax.axis_index("s")
    wid = s * 2 + c

    z16 = jnp.zeros((16,), jnp.float32)

    def _zero_zb(i, carry):
        zb_v[i, pl.ds(0, 16)] = z16
        return carry

    lax.fori_loop(0, ZR, _zero_zb, 0)

    def _zero_msg(i, carry):
        msg_v[i, pl.ds(0, 16)] = z16
        return carry

    lax.fori_loop(0, CH, _zero_msg, 0)
    for i in range(ZCOPIES):
        pltpu.sync_copy(zb_v, acc_sh.at[pl.ds(s * ROWS_PER_TILE + i * ZR, ZR)])
    plsc.subcore_barrier()

    nch = 390 + jnp.where(wid < NCHUNK - 390 * NW, 1, 0)
    iota = lax.iota(jnp.int32, 16)

    def _chunk(j, carry):
        base = (wid + j * NW) * CH
        pltpu.sync_copy(src_hbm.at[pl.ds(base, CH)], src_v)
        pltpu.sync_copy(dst_hbm.at[pl.ds(base, CH)], dst_v)
        pltpu.sync_copy(e2_hbm.at[pl.ds(base, CH)], e2_v)
        cp1 = pltpu.async_copy(t2_hbm.at[dst_v], td_v, sem1)
        cp2 = pltpu.async_copy(t2_hbm.at[src_v], ts_v, sem2)
        cp1.wait()
        cp2.wait()
        for g in range(CH // 16):
            lanes = iota + (g * 16)
            q2 = plsc.load_gather(td_v, [lanes, _full(0)])
            k2 = plsc.load_gather(ts_v, [lanes, _full(1)])
            v2 = plsc.load_gather(ts_v, [lanes, _full(2)])
            e2 = e2_v[pl.ds(g * 16, 16)]
            alpha = jnp.clip(q2 * (k2 + e2), -75.0, 75.0)
            w = jnp.exp(alpha)
            plsc.store_scatter(msg_v, [lanes, _full(0)], w * (v2 + e2))
            plsc.store_scatter(msg_v, [lanes, _full(1)], w)
        pltpu.sync_copy(msg_v, acc_sh.at[dst_v], add=True)
        return carry

    lax.fori_loop(0, nch, _chunk, 0)
    plsc.subcore_barrier()
    r0 = s * ROWS_PER_TILE
    pltpu.sync_copy(acc_sh.at[pl.ds(r0, ROWS_PER_TILE)],
                    out_hbm.at[c, pl.ds(r0, ROWS_PER_TILE)])


def kernel(x, edge_index, edge_attr,
           Wq1, bq1, Wk1, bk1, Wv1, bv1, We1, Ws1, bs1,
           Wq2, bq2, Wk2, bk2, Wv2, bv2, We2, Ws2, bs2):
    src = edge_index[0].astype(jnp.int32)
    dst = edge_index[1].astype(jnp.int32)
    ea = edge_attr.astype(jnp.float32)

    q1 = x @ Wq1.T + bq1
    k1 = x @ Wk1.T + bk1
    v1 = x @ Wv1.T + bv1
    e1 = ea @ We1.T
    kv = jnp.concatenate([k1, v1], axis=1)

    acc = _edge_pass1(q1, kv, src, dst, e1)
    a = acc[0, :NN] + acc[1, :NN]
    accv = a[:, 0:16]
    den = a[:, 16:17]
    den = jnp.where(den == 0.0, 1.0, den)
    h = jax.nn.relu(accv / den + x @ Ws1.T + bs1)

    q2 = h @ Wq2.T + bq2
    k2 = h @ Wk2.T + bk2
    v2 = h @ Wv2.T + bv2
    e2 = (ea @ We2.T).reshape(-1)
    t2 = jnp.concatenate([q2, k2, v2, jnp.zeros((NN, 13), jnp.float32)],
                         axis=1)

    acc2 = _edge_pass2(t2, src, dst, e2)
    a2 = acc2[0, :NN] + acc2[1, :NN]
    num = a2[:, 0:1]
    den2 = a2[:, 1:2]
    den2 = jnp.where(den2 == 0.0, 1.0, den2)
    return jax.nn.sigmoid(num / den2 + h @ Ws2.T + bs2)


# speedup vs baseline: 27.4422x; 1.3109x over previous
import functools

import jax
import jax.numpy as jnp
from jax import lax
from jax.experimental import pallas as pl
from jax.experimental.pallas import tpu as pltpu
from jax.experimental.pallas import tpu_sc as plsc

NN = 50000
EE = 1600000
CH = 128
NCHUNK = EE // CH
NW = 32
NB = 2
NG = 196
ROWS_PER_TILE = 3128
NPAD = 16 * ROWS_PER_TILE
ZR = 184
ZCOPIES = ROWS_PER_TILE // ZR

_mesh = plsc.VectorSubcoreMesh(core_axis_name="c", subcore_axis_name="s")
_params = pltpu.CompilerParams(needs_layout_passes=False,
                               use_tc_tiling_on_sc=False,
                               internal_scratch_in_bytes=131072)


def _full(val):
    return jnp.full((16,), val, jnp.int32)


def _scratch1():
    per_buf = [
        pltpu.VMEM((CH,), jnp.int32),
        pltpu.VMEM((CH,), jnp.int32),
        pltpu.VMEM((CH, 16), jnp.float32),
        pltpu.VMEM((CH, 16), jnp.float32),
        pltpu.VMEM((CH, 32), jnp.float32),
        pltpu.VMEM((CH, 32), jnp.float32),
        pltpu.SemaphoreType.DMA,
        pltpu.SemaphoreType.DMA,
        pltpu.SemaphoreType.DMA,
    ]
    return per_buf * NB + [
        pltpu.VMEM((ZR, 32), jnp.float32),
        pltpu.VMEM_SHARED((NPAD, 32), jnp.float32),
    ]


@functools.partial(
    pl.kernel,
    out_type=jax.ShapeDtypeStruct((2, NPAD, 32), jnp.float32),
    mesh=_mesh,
    scratch_types=_scratch1(),
    compiler_params=_params,
)
def _edge_pass1(q_hbm, kv_hbm, src_hbm, dst_hbm, e_hbm, out_hbm, *scr):
    bufs = [scr[9 * b:9 * (b + 1)] for b in range(NB)]
    zb_v = scr[9 * NB]
    acc_sh = scr[9 * NB + 1]

    c = lax.axis_index("c")
    s = lax.axis_index("s")
    wid = s * 2 + c

    z16 = jnp.zeros((16,), jnp.float32)

    def _zero_zb(i, carry):
        zb_v[i, pl.ds(0, 16)] = z16
        zb_v[i, pl.ds(16, 16)] = z16
        return carry

    lax.fori_loop(0, ZR, _zero_zb, 0)
    for b in range(NB):
        msg_v = bufs[b][5]

        def _zero_msg(i, carry, msg_v=msg_v):
            msg_v[i, pl.ds(0, 16)] = z16
            msg_v[i, pl.ds(16, 16)] = z16
            return carry

        lax.fori_loop(0, CH, _zero_msg, 0)
    for i in range(ZCOPIES):
        pltpu.sync_copy(zb_v, acc_sh.at[pl.ds(s * ROWS_PER_TILE + i * ZR, ZR)])
    plsc.subcore_barrier()

    iota = lax.iota(jnp.int32, 16)

    def _group(i, carry):
        for b in range(NB):
            src_v, dst_v, e_v, q_v, kv_v, msg_v, semi, semg, semsc = bufs[b]
            j = i * NB + b
            prev_ok = jnp.logical_and(
                i > 0, wid + (j - NB) * NW < NCHUNK)

            @pl.when(prev_ok)
            def _(msg_v=msg_v, dst_v=dst_v, semsc=semsc):
                pltpu.make_async_copy(msg_v, acc_sh.at[dst_v], semsc).wait()

            @pl.when(wid + j * NW < NCHUNK)
            def _(src_v=src_v, dst_v=dst_v, e_v=e_v, semi=semi, j=j):
                base = (wid + j * NW) * CH
                pltpu.async_copy(src_hbm.at[pl.ds(base, CH)], src_v, semi)
                pltpu.async_copy(dst_hbm.at[pl.ds(base, CH)], dst_v, semi)
                pltpu.async_copy(e_hbm.at[pl.ds(base, CH)], e_v, semi)

        for b in range(NB):
            src_v, dst_v, e_v, q_v, kv_v, msg_v, semi, semg, semsc = bufs[b]
            j = i * NB + b

            @pl.when(wid + j * NW < NCHUNK)
            def _(src_v=src_v, dst_v=dst_v, e_v=e_v, q_v=q_v, kv_v=kv_v,
                  semi=semi, semg=semg):
                pltpu.make_async_copy(src_hbm.at[pl.ds(0, CH)], src_v,
                                      semi).wait()
                pltpu.make_async_copy(dst_hbm.at[pl.ds(0, CH)], dst_v,
                                      semi).wait()
                pltpu.make_async_copy(e_hbm.at[pl.ds(0, CH)], e_v,
                                      semi).wait()
                pltpu.async_copy(q_hbm.at[dst_v], q_v, semg)
                pltpu.async_copy(kv_hbm.at[src_v], kv_v, semg)

        for b in range(NB):
            src_v, dst_v, e_v, q_v, kv_v, msg_v, semi, semg, semsc = bufs[b]
            j = i * NB + b
            ok = wid + j * NW < NCHUNK

            @pl.when(ok)
            def _(dst_v=dst_v, src_v=src_v, q_v=q_v, kv_v=kv_v, semg=semg):
                pltpu.make_async_copy(q_hbm.at[dst_v], q_v, semg).wait()
                pltpu.make_async_copy(kv_hbm.at[src_v], kv_v, semg).wait()

            def _grp(g, carry, e_v=e_v, q_v=q_v, kv_v=kv_v, msg_v=msg_v):
                lanes = iota + g * 16
                acc = jnp.zeros((16,), jnp.float32)
                for f in range(16):
                    qf = plsc.load_gather(q_v, [lanes, _full(f)])
                    kf = plsc.load_gather(kv_v, [lanes, _full(f)])
                    ef = plsc.load_gather(e_v, [lanes, _full(f)])
                    acc = acc + qf * (kf + ef)
                alpha = jnp.clip(acc * 0.25, -75.0, 75.0)
                w = jnp.exp(alpha)
                for f in range(16):
                    vf = plsc.load_gather(kv_v, [lanes, _full(16 + f)])
                    ef = plsc.load_gather(e_v, [lanes, _full(f)])
                    plsc.store_scatter(msg_v, [lanes, _full(f)],
                                       w * (vf + ef))
                plsc.store_scatter(msg_v, [lanes, _full(16)], w)
                return carry

            lax.fori_loop(0, CH // 16, _grp, 0)

            @pl.when(ok)
            def _(msg_v=msg_v, dst_v=dst_v, semsc=semsc):
                pltpu.async_copy(msg_v, acc_sh.at[dst_v], semsc, add=True)

        return carry

    lax.fori_loop(0, NG, _group, 0)

    for b in range(NB):
        src_v, dst_v, e_v, q_v, kv_v, msg_v, semi, semg, semsc = bufs[b]
        jl = (NG - 1) * NB + b

        @pl.when(wid + jl * NW < NCHUNK)
        def _(msg_v=msg_v, dst_v=dst_v, semsc=semsc):
            pltpu.make_async_copy(msg_v, acc_sh.at[dst_v], semsc).wait()

    plsc.subcore_barrier()
    r0 = s * ROWS_PER_TILE
    pltpu.sync_copy(acc_sh.at[pl.ds(r0, ROWS_PER_TILE)],
                    out_hbm.at[c, pl.ds(r0, ROWS_PER_TILE)])


def _scratch2():
    per_buf = [
        pltpu.VMEM((CH,), jnp.int32),
        pltpu.VMEM((CH,), jnp.int32),
        pltpu.VMEM((CH,), jnp.float32),
        pltpu.VMEM((CH, 16), jnp.float32),
        pltpu.VMEM((CH, 16), jnp.float32),
        pltpu.VMEM((CH, 16), jnp.float32),
        pltpu.SemaphoreType.DMA,
        pltpu.SemaphoreType.DMA,
        pltpu.SemaphoreType.DMA,
    ]
    return per_buf * NB + [
        pltpu.VMEM((ZR, 16), jnp.float32),
        pltpu.VMEM_SHARED((NPAD, 16), jnp.float32),
    ]


@functools.partial(
    pl.kernel,
    out_type=jax.ShapeDtypeStruct((2, NPAD, 16), jnp.float32),
    mesh=_mesh,
    scratch_types=_scratch2(),
    compiler_params=_params,
)
def _edge_pass2(t2_hbm, src_hbm, dst_hbm, e2_hbm, out_hbm, *scr):
    bufs = [scr[9 * b:9 * (b + 1)] for b in range(NB)]
    zb_v = scr[9 * NB]
    acc_sh = scr[9 * NB + 1]

    c = lax.axis_index("c")
    s = lax.axis_index("s")
    wid = s * 2 + c

    z16 = jnp.zeros((16,), jnp.float32)

    def _zero_zb(i, carry):
        zb_v[i, pl.ds(0, 16)] = z16
        return carry

    lax.fori_loop(0, ZR, _zero_zb, 0)
    for b in range(NB):
        msg_v = bufs[b][5]

        def _zero_msg(i, carry, msg_v=msg_v):
            msg_v[i, pl.ds(0, 16)] = z16
            return carry

        lax.fori_loop(0, CH, _zero_msg, 0)
    for i in range(ZCOPIES):
        pltpu.sync_copy(zb_v, acc_sh.at[pl.ds(s * ROWS_PER_TILE + i * ZR, ZR)])
    plsc.subcore_barrier()

    iota = lax.iota(jnp.int32, 16)

    def _group(i, carry):
        for b in range(NB):
            src_v, dst_v, e2_v, ts_v, td_v, msg_v, semi, semg, semsc = bufs[b]
            j = i * NB + b
            prev_ok = jnp.logical_and(
                i > 0, wid + (j - NB) * NW < NCHUNK)

            @pl.when(prev_ok)
            def _(msg_v=msg_v, dst_v=dst_v, semsc=semsc):
                pltpu.make_async_copy(msg_v, acc_sh.at[dst_v], semsc).wait()

            @pl.when(wid + j * NW < NCHUNK)
            def _(src_v=src_v, dst_v=dst_v, e2_v=e2_v, semi=semi, j=j):
                base = (wid + j * NW) * CH
                pltpu.async_copy(src_hbm.at[pl.ds(base, CH)], src_v, semi)
                pltpu.async_copy(dst_hbm.at[pl.ds(base, CH)], dst_v, semi)
                pltpu.async_copy(e2_hbm.at[pl.ds(base, CH)], e2_v, semi)

        for b in range(NB):
            src_v, dst_v, e2_v, ts_v, td_v, msg_v, semi, semg, semsc = bufs[b]
            j = i * NB + b

            @pl.when(wid + j * NW < NCHUNK)
            def _(src_v=src_v, dst_v=dst_v, e2_v=e2_v, ts_v=ts_v, td_v=td_v,
                  semi=semi, semg=semg):
                pltpu.make_async_copy(src_hbm.at[pl.ds(0, CH)], src_v,
                                      semi).wait()
                pltpu.make_async_copy(dst_hbm.at[pl.ds(0, CH)], dst_v,
                                      semi).wait()
                pltpu.make_async_copy(e2_hbm.at[pl.ds(0, CH)], e2_v,
                                      semi).wait()
                pltpu.async_copy(t2_hbm.at[dst_v], td_v, semg)
                pltpu.async_copy(t2_hbm.at[src_v], ts_v, semg)

        for b in range(NB):
            src_v, dst_v, e2_v, ts_v, td_v, msg_v, semi, semg, semsc = bufs[b]
            j = i * NB + b
            ok = wid + j * NW < NCHUNK

            @pl.when(ok)
            def _(src_v=src_v, dst_v=dst_v, ts_v=ts_v, td_v=td_v, semg=semg):
                pltpu.make_async_copy(t2_hbm.at[dst_v], td_v, semg).wait()
                pltpu.make_async_copy(t2_hbm.at[src_v], ts_v, semg).wait()

            def _grp(g, carry, e2_v=e2_v, ts_v=ts_v, td_v=td_v, msg_v=msg_v):
                lanes = iota + g * 16
                q2 = plsc.load_gather(td_v, [lanes, _full(0)])
                k2 = plsc.load_gather(ts_v, [lanes, _full(1)])
                v2 = plsc.load_gather(ts_v, [lanes, _full(2)])
                e2 = e2_v[pl.ds(g * 16, 16)]
                alpha = jnp.clip(q2 * (k2 + e2), -75.0, 75.0)
                w = jnp.exp(alpha)
                plsc.store_scatter(msg_v, [lanes, _full(0)], w * (v2 + e2))
                plsc.store_scatter(msg_v, [lanes, _full(1)], w)
                return carry

            lax.fori_loop(0, CH // 16, _grp, 0)

            @pl.when(ok)
            def _(msg_v=msg_v, dst_v=dst_v, semsc=semsc):
                pltpu.async_copy(msg_v, acc_sh.at[dst_v], semsc, add=True)

        return carry

    lax.fori_loop(0, NG, _group, 0)

    for b in range(NB):
        src_v, dst_v, e2_v, ts_v, td_v, msg_v, semi, semg, semsc = bufs[b]
        jl = (NG - 1) * NB + b

        @pl.when(wid + jl * NW < NCHUNK)
        def _(msg_v=msg_v, dst_v=dst_v, semsc=semsc):
            pltpu.make_async_copy(msg_v, acc_sh.at[dst_v], semsc).wait()

    plsc.subcore_barrier()
    r0 = s * ROWS_PER_TILE
    pltpu.sync_copy(acc_sh.at[pl.ds(r0, ROWS_PER_TILE)],
                    out_hbm.at[c, pl.ds(r0, ROWS_PER_TILE)])


def kernel(x, edge_index, edge_attr,
           Wq1, bq1, Wk1, bk1, Wv1, bv1, We1, Ws1, bs1,
           Wq2, bq2, Wk2, bk2, Wv2, bv2, We2, Ws2, bs2):
    src = edge_index[0].astype(jnp.int32)
    dst = edge_index[1].astype(jnp.int32)
    ea = edge_attr.astype(jnp.float32)

    q1 = x @ Wq1.T + bq1
    k1 = x @ Wk1.T + bk1
    v1 = x @ Wv1.T + bv1
    e1 = ea @ We1.T
    kv = jnp.concatenate([k1, v1], axis=1)

    acc = _edge_pass1(q1, kv, src, dst, e1)
    a = acc[0, :NN] + acc[1, :NN]
    accv = a[:, 0:16]
    den = a[:, 16:17]
    den = jnp.where(den == 0.0, 1.0, den)
    h = jax.nn.relu(accv / den + x @ Ws1.T + bs1)

    q2 = h @ Wq2.T + bq2
    k2 = h @ Wk2.T + bk2
    v2 = h @ Wv2.T + bv2
    e2 = (ea @ We2.T).reshape(-1)
    t2 = jnp.concatenate([q2, k2, v2, jnp.zeros((NN, 13), jnp.float32)],
                         axis=1)

    acc2 = _edge_pass2(t2, src, dst, e2)
    a2 = acc2[0, :NN] + acc2[1, :NN]
    num = a2[:, 0:1]
    den2 = a2[:, 1:2]
    den2 = jnp.where(den2 == 0.0, 1.0, den2)
    return jax.nn.sigmoid(num / den2 + h @ Ws2.T + bs2)


# speedup vs baseline: 32.5715x; 1.1869x over previous
import functools

import jax
import jax.numpy as jnp
from jax import lax
from jax.experimental import pallas as pl
from jax.experimental.pallas import tpu as pltpu
from jax.experimental.pallas import tpu_sc as plsc

NN = 50000
EE = 1600000
CH = 128
NCHUNK = EE // CH
NW = 32
NB = 4
NG = 98
ROWS_PER_TILE = 3128
NPAD = 16 * ROWS_PER_TILE
ZR = 184
ZCOPIES = ROWS_PER_TILE // ZR

_mesh = plsc.VectorSubcoreMesh(core_axis_name="c", subcore_axis_name="s")
_params = pltpu.CompilerParams(needs_layout_passes=False,
                               use_tc_tiling_on_sc=False,
                               internal_scratch_in_bytes=131072)


def _full(val):
    return jnp.full((16,), val, jnp.int32)


def _scratch1():
    per_buf = [
        pltpu.VMEM((CH,), jnp.int32),
        pltpu.VMEM((CH,), jnp.int32),
        pltpu.VMEM((CH, 16), jnp.float32),
        pltpu.VMEM((CH, 16), jnp.float32),
        pltpu.VMEM((CH, 32), jnp.float32),
        pltpu.VMEM((CH, 24), jnp.float32),
        pltpu.SemaphoreType.DMA,
        pltpu.SemaphoreType.DMA,
        pltpu.SemaphoreType.DMA,
    ]
    return per_buf * NB + [
        pltpu.VMEM_SHARED((NPAD, 24), jnp.float32),
    ]


@functools.partial(
    pl.kernel,
    out_type=jax.ShapeDtypeStruct((2, NPAD, 24), jnp.float32),
    mesh=_mesh,
    scratch_types=_scratch1(),
    compiler_params=_params,
)
def _edge_pass1(q_hbm, kv_hbm, src_hbm, dst_hbm, e_hbm, out_hbm, *scr):
    bufs = [scr[9 * b:9 * (b + 1)] for b in range(NB)]
    acc_sh = scr[9 * NB]

    c = lax.axis_index("c")
    s = lax.axis_index("s")
    wid = s * 2 + c

    z16 = jnp.zeros((16,), jnp.float32)
    z8 = jnp.zeros((16,), jnp.float32)
    for b in range(NB):
        msg_v = bufs[b][5]

        def _zero_msg(i, carry, msg_v=msg_v):
            msg_v[i, pl.ds(0, 16)] = z16
            msg_v[i, pl.ds(8, 16)] = z16
            return carry

        lax.fori_loop(0, CH, _zero_msg, 0)
    zmsg = bufs[0][5]
    for i in range(24):
        pltpu.sync_copy(zmsg, acc_sh.at[pl.ds(s * ROWS_PER_TILE + i * CH, CH)])
    pltpu.sync_copy(zmsg.at[pl.ds(0, 56)],
                    acc_sh.at[pl.ds(s * ROWS_PER_TILE + 24 * CH, 56)])
    plsc.subcore_barrier()

    iota = lax.iota(jnp.int32, 16)

    def _group(i, carry):
        for b in range(NB):
            src_v, dst_v, e_v, q_v, kv_v, msg_v, semi, semg, semsc = bufs[b]
            j = i * NB + b
            prev_ok = jnp.logical_and(
                i > 0, wid + (j - NB) * NW < NCHUNK)

            @pl.when(prev_ok)
            def _(msg_v=msg_v, dst_v=dst_v, semsc=semsc):
                pltpu.make_async_copy(msg_v, acc_sh.at[dst_v], semsc).wait()

            @pl.when(wid + j * NW < NCHUNK)
            def _(src_v=src_v, dst_v=dst_v, e_v=e_v, semi=semi, j=j):
                base = (wid + j * NW) * CH
                pltpu.async_copy(src_hbm.at[pl.ds(base, CH)], src_v, semi)
                pltpu.async_copy(dst_hbm.at[pl.ds(base, CH)], dst_v, semi)
                pltpu.async_copy(e_hbm.at[pl.ds(base, CH)], e_v, semi)

        for b in range(NB):
            src_v, dst_v, e_v, q_v, kv_v, msg_v, semi, semg, semsc = bufs[b]
            j = i * NB + b

            @pl.when(wid + j * NW < NCHUNK)
            def _(src_v=src_v, dst_v=dst_v, e_v=e_v, q_v=q_v, kv_v=kv_v,
                  semi=semi, semg=semg):
                pltpu.make_async_copy(src_hbm.at[pl.ds(0, CH)], src_v,
                                      semi).wait()
                pltpu.make_async_copy(dst_hbm.at[pl.ds(0, CH)], dst_v,
                                      semi).wait()
                pltpu.make_async_copy(e_hbm.at[pl.ds(0, CH)], e_v,
                                      semi).wait()
                pltpu.async_copy(q_hbm.at[dst_v], q_v, semg)
                pltpu.async_copy(kv_hbm.at[src_v], kv_v, semg)

        for b in range(NB):
            src_v, dst_v, e_v, q_v, kv_v, msg_v, semi, semg, semsc = bufs[b]
            j = i * NB + b
            ok = wid + j * NW < NCHUNK

            @pl.when(ok)
            def _(dst_v=dst_v, src_v=src_v, q_v=q_v, kv_v=kv_v, semg=semg):
                pltpu.make_async_copy(q_hbm.at[dst_v], q_v, semg).wait()
                pltpu.make_async_copy(kv_hbm.at[src_v], kv_v, semg).wait()

            def _grp(g, carry, e_v=e_v, q_v=q_v, kv_v=kv_v, msg_v=msg_v):
                lanes = iota + g * 16
                acc = jnp.zeros((16,), jnp.float32)
                for f in range(16):
                    qf = plsc.load_gather(q_v, [lanes, _full(f)])
                    kf = plsc.load_gather(kv_v, [lanes, _full(f)])
                    ef = plsc.load_gather(e_v, [lanes, _full(f)])
                    acc = acc + qf * (kf + ef)
                alpha = jnp.clip(acc * 0.25, -75.0, 75.0)
                w = jnp.exp(alpha)
                for f in range(16):
                    vf = plsc.load_gather(kv_v, [lanes, _full(16 + f)])
                    ef = plsc.load_gather(e_v, [lanes, _full(f)])
                    plsc.store_scatter(msg_v, [lanes, _full(f)],
                                       w * (vf + ef))
                plsc.store_scatter(msg_v, [lanes, _full(16)], w)
                return carry

            lax.fori_loop(0, CH // 16, _grp, 0)

            @pl.when(ok)
            def _(msg_v=msg_v, dst_v=dst_v, semsc=semsc):
                pltpu.async_copy(msg_v, acc_sh.at[dst_v], semsc, add=True)

        return carry

    lax.fori_loop(0, NG, _group, 0)

    for b in range(NB):
        src_v, dst_v, e_v, q_v, kv_v, msg_v, semi, semg, semsc = bufs[b]
        jl = (NG - 1) * NB + b

        @pl.when(wid + jl * NW < NCHUNK)
        def _(msg_v=msg_v, dst_v=dst_v, semsc=semsc):
            pltpu.make_async_copy(msg_v, acc_sh.at[dst_v], semsc).wait()

    plsc.subcore_barrier()
    r0 = s * ROWS_PER_TILE
    pltpu.sync_copy(acc_sh.at[pl.ds(r0, ROWS_PER_TILE)],
                    out_hbm.at[c, pl.ds(r0, ROWS_PER_TILE)])


def _scratch2():
    per_buf = [
        pltpu.VMEM((CH,), jnp.int32),
        pltpu.VMEM((CH,), jnp.int32),
        pltpu.VMEM((CH,), jnp.float32),
        pltpu.VMEM((CH, 16), jnp.float32),
        pltpu.VMEM((CH, 16), jnp.float32),
        pltpu.VMEM((CH, 16), jnp.float32),
        pltpu.SemaphoreType.DMA,
        pltpu.SemaphoreType.DMA,
        pltpu.SemaphoreType.DMA,
    ]
    return per_buf * NB + [
        pltpu.VMEM_SHARED((NPAD, 16), jnp.float32),
    ]


@functools.partial(
    pl.kernel,
    out_type=jax.ShapeDtypeStruct((2, NPAD, 16), jnp.float32),
    mesh=_mesh,
    scratch_types=_scratch2(),
    compiler_params=_params,
)
def _edge_pass2(t2_hbm, src_hbm, dst_hbm, e2_hbm, out_hbm, *scr):
    bufs = [scr[9 * b:9 * (b + 1)] for b in range(NB)]
    acc_sh = scr[9 * NB]

    c = lax.axis_index("c")
    s = lax.axis_index("s")
    wid = s * 2 + c

    z16 = jnp.zeros((16,), jnp.float32)
    for b in range(NB):
        msg_v = bufs[b][5]

        def _zero_msg(i, carry, msg_v=msg_v):
            msg_v[i, pl.ds(0, 16)] = z16
            return carry

        lax.fori_loop(0, CH, _zero_msg, 0)
    zmsg = bufs[0][5]
    for i in range(24):
        pltpu.sync_copy(zmsg, acc_sh.at[pl.ds(s * ROWS_PER_TILE + i * CH, CH)])
    pltpu.sync_copy(zmsg.at[pl.ds(0, 56)],
                    acc_sh.at[pl.ds(s * ROWS_PER_TILE + 24 * CH, 56)])
    plsc.subcore_barrier()

    iota = lax.iota(jnp.int32, 16)

    def _group(i, carry):
        for b in range(NB):
            src_v, dst_v, e2_v, ts_v, td_v, msg_v, semi, semg, semsc = bufs[b]
            j = i * NB + b
            prev_ok = jnp.logical_and(
                i > 0, wid + (j - NB) * NW < NCHUNK)

            @pl.when(prev_ok)
            def _(msg_v=msg_v, dst_v=dst_v, semsc=semsc):
                pltpu.make_async_copy(msg_v, acc_sh.at[dst_v], semsc).wait()

            @pl.when(wid + j * NW < NCHUNK)
            def _(src_v=src_v, dst_v=dst_v, e2_v=e2_v, semi=semi, j=j):
                base = (wid + j * NW) * CH
                pltpu.async_copy(src_hbm.at[pl.ds(base, CH)], src_v, semi)
                pltpu.async_copy(dst_hbm.at[pl.ds(base, CH)], dst_v, semi)
                pltpu.async_copy(e2_hbm.at[pl.ds(base, CH)], e2_v, semi)

        for b in range(NB):
            src_v, dst_v, e2_v, ts_v, td_v, msg_v, semi, semg, semsc = bufs[b]
            j = i * NB + b

            @pl.when(wid + j * NW < NCHUNK)
            def _(src_v=src_v, dst_v=dst_v, e2_v=e2_v, ts_v=ts_v, td_v=td_v,
                  semi=semi, semg=semg):
                pltpu.make_async_copy(src_hbm.at[pl.ds(0, CH)], src_v,
                                      semi).wait()
                pltpu.make_async_copy(dst_hbm.at[pl.ds(0, CH)], dst_v,
                                      semi).wait()
                pltpu.make_async_copy(e2_hbm.at[pl.ds(0, CH)], e2_v,
                                      semi).wait()
                pltpu.async_copy(t2_hbm.at[dst_v], td_v, semg)
                pltpu.async_copy(t2_hbm.at[src_v], ts_v, semg)

        for b in range(NB):
            src_v, dst_v, e2_v, ts_v, td_v, msg_v, semi, semg, semsc = bufs[b]
            j = i * NB + b
            ok = wid + j * NW < NCHUNK

            @pl.when(ok)
            def _(src_v=src_v, dst_v=dst_v, ts_v=ts_v, td_v=td_v, semg=semg):
                pltpu.make_async_copy(t2_hbm.at[dst_v], td_v, semg).wait()
                pltpu.make_async_copy(t2_hbm.at[src_v], ts_v, semg).wait()

            def _grp(g, carry, e2_v=e2_v, ts_v=ts_v, td_v=td_v, msg_v=msg_v):
                lanes = iota + g * 16
                q2 = plsc.load_gather(td_v, [lanes, _full(0)])
                k2 = plsc.load_gather(ts_v, [lanes, _full(1)])
                v2 = plsc.load_gather(ts_v, [lanes, _full(2)])
                e2 = e2_v[pl.ds(g * 16, 16)]
                alpha = jnp.clip(q2 * (k2 + e2), -75.0, 75.0)
                w = jnp.exp(alpha)
                plsc.store_scatter(msg_v, [lanes, _full(0)], w * (v2 + e2))
                plsc.store_scatter(msg_v, [lanes, _full(1)], w)
                return carry

            lax.fori_loop(0, CH // 16, _grp, 0)

            @pl.when(ok)
            def _(msg_v=msg_v, dst_v=dst_v, semsc=semsc):
                pltpu.async_copy(msg_v, acc_sh.at[dst_v], semsc, add=True)

        return carry

    lax.fori_loop(0, NG, _group, 0)

    for b in range(NB):
        src_v, dst_v, e2_v, ts_v, td_v, msg_v, semi, semg, semsc = bufs[b]
        jl = (NG - 1) * NB + b

        @pl.when(wid + jl * NW < NCHUNK)
        def _(msg_v=msg_v, dst_v=dst_v, semsc=semsc):
            pltpu.make_async_copy(msg_v, acc_sh.at[dst_v], semsc).wait()

    plsc.subcore_barrier()
    r0 = s * ROWS_PER_TILE
    pltpu.sync_copy(acc_sh.at[pl.ds(r0, ROWS_PER_TILE)],
                    out_hbm.at[c, pl.ds(r0, ROWS_PER_TILE)])


def kernel(x, edge_index, edge_attr,
           Wq1, bq1, Wk1, bk1, Wv1, bv1, We1, Ws1, bs1,
           Wq2, bq2, Wk2, bk2, Wv2, bv2, We2, Ws2, bs2):
    src = edge_index[0].astype(jnp.int32)
    dst = edge_index[1].astype(jnp.int32)
    ea = edge_attr.astype(jnp.float32)

    q1 = x @ Wq1.T + bq1
    k1 = x @ Wk1.T + bk1
    v1 = x @ Wv1.T + bv1
    e1 = ea @ We1.T
    kv = jnp.concatenate([k1, v1], axis=1)

    acc = _edge_pass1(q1, kv, src, dst, e1)
    a = acc[0, :NN] + acc[1, :NN]
    accv = a[:, 0:16]
    den = a[:, 16:17]
    den = jnp.where(den == 0.0, 1.0, den)
    h = jax.nn.relu(accv / den + x @ Ws1.T + bs1)

    q2 = h @ Wq2.T + bq2
    k2 = h @ Wk2.T + bk2
    v2 = h @ Wv2.T + bv2
    e2 = (ea @ We2.T).reshape(-1)
    t2 = jnp.concatenate([q2, k2, v2, jnp.zeros((NN, 13), jnp.float32)],
                         axis=1)

    acc2 = _edge_pass2(t2, src, dst, e2)
    a2 = acc2[0, :NN] + acc2[1, :NN]
    num = a2[:, 0:1]
    den2 = a2[:, 1:2]
    den2 = jnp.where(den2 == 0.0, 1.0, den2)
    return jax.nn.sigmoid(num / den2 + h @ Ws2.T + bs2)
